# Initial kernel scaffold; baseline (speedup 1.0000x reference)
import jax
import jax.numpy as jnp
from jax.experimental import pallas as pl


def kernel(x, edge_index, W1, b1, W2, b2, W3, b3):
    raise NotImplementedError("write your pallas kernel here")


# speedup vs baseline: 5.5117x; 5.5117x over previous
import jax
import jax.numpy as jnp
from jax import lax
from jax.experimental import pallas as pl
from jax.experimental.pallas import tpu as pltpu
from jax.experimental.pallas import tpu_sc as plsc

N = 10000
E = 320000
D = 128
NCLS = 47

NC = 2
NS = 16
NW = NC * NS
E_PER_W = E // NW
CH = 80
NCHUNK = E_PER_W // CH
NP = 10240
RPW = NP // NS
NRB = RPW // CH
L = 16

_sc_mesh = plsc.VectorSubcoreMesh(
    core_axis_name="c", subcore_axis_name="s", num_cores=NC, num_subcores=NS)


def _seg_body_deg(src_hbm, dst_hbm, feat_hbm, zrow_hbm, zdeg_hbm,
                  acc_out, deg_out, src_v, dst_v, rows_v, deg_v, acc_sh,
                  sem):
  c = lax.axis_index("c")
  s = lax.axis_index("s")
  base = (c * NS + s) * E_PER_W
  r0 = s * RPW

  pltpu.sync_copy(zrow_hbm, rows_v)
  pltpu.sync_copy(zdeg_hbm, deg_v)
  for j in range(NRB):
    pltpu.sync_copy(rows_v, acc_sh.at[pl.ds(r0 + j * CH, CH)])
  plsc.subcore_barrier()

  one16 = jnp.full((L,), 1.0, jnp.float32)

  def step(i, carry):
    off = base + i * CH
    pltpu.sync_copy(src_hbm.at[pl.ds(off, CH)], src_v)
    pltpu.sync_copy(dst_hbm.at[pl.ds(off, CH)], dst_v)
    pltpu.async_copy(feat_hbm.at[src_v], rows_v, sem).wait()
    pltpu.sync_copy(rows_v, acc_sh.at[dst_v], add=True)
    for k in range(CH // L):
      idx = dst_v[pl.ds(k * L, L)]
      plsc.addupdate_scatter(deg_v, [idx], one16)
    return carry

  lax.fori_loop(0, NCHUNK, step, 0)
  plsc.subcore_barrier()

  for j in range(NRB):
    b0 = r0 + j * CH
    pltpu.sync_copy(acc_sh.at[pl.ds(b0, CH)], rows_v)
    pltpu.sync_copy(rows_v, acc_out.at[c, pl.ds(b0, CH)])
  pltpu.sync_copy(deg_v, deg_out.at[c, s])


def _seg_body_plain(src_hbm, dst_hbm, feat_hbm, zrow_hbm, acc_out, src_v,
                    dst_v, rows_v, acc_sh, sem):
  c = lax.axis_index("c")
  s = lax.axis_index("s")
  base = (c * NS + s) * E_PER_W
  r0 = s * RPW

  pltpu.sync_copy(zrow_hbm, rows_v)
  for j in range(NRB):
    pltpu.sync_copy(rows_v, acc_sh.at[pl.ds(r0 + j * CH, CH)])
  plsc.subcore_barrier()

  def step(i, carry):
    off = base + i * CH
    pltpu.sync_copy(src_hbm.at[pl.ds(off, CH)], src_v)
    pltpu.sync_copy(dst_hbm.at[pl.ds(off, CH)], dst_v)
    pltpu.async_copy(feat_hbm.at[src_v], rows_v, sem).wait()
    pltpu.sync_copy(rows_v, acc_sh.at[dst_v], add=True)
    return carry

  lax.fori_loop(0, NCHUNK, step, 0)
  plsc.subcore_barrier()

  for j in range(NRB):
    b0 = r0 + j * CH
    pltpu.sync_copy(acc_sh.at[pl.ds(b0, CH)], rows_v)
    pltpu.sync_copy(rows_v, acc_out.at[c, pl.ds(b0, CH)])


_seg_sum_deg = pl.kernel(
    _seg_body_deg,
    out_type=(jax.ShapeDtypeStruct((NC, NP, D), jnp.float32),
              jax.ShapeDtypeStruct((NC, NS, NP), jnp.float32)),
    mesh=_sc_mesh,
    scratch_types=[
        pltpu.VMEM((CH,), jnp.int32),
        pltpu.VMEM((CH,), jnp.int32),
        pltpu.VMEM((CH, D), jnp.float32),
        pltpu.VMEM((NP,), jnp.float32),
        pltpu.VMEM_SHARED((NP, D), jnp.float32),
        pltpu.SemaphoreType.DMA,
    ],
    compiler_params=pltpu.CompilerParams(needs_layout_passes=False),
    name="sc_seg_sum_deg",
)

_seg_sum_plain = pl.kernel(
    _seg_body_plain,
    out_type=jax.ShapeDtypeStruct((NC, NP, D), jnp.float32),
    mesh=_sc_mesh,
    scratch_types=[
        pltpu.VMEM((CH,), jnp.int32),
        pltpu.VMEM((CH,), jnp.int32),
        pltpu.VMEM((CH, D), jnp.float32),
        pltpu.VMEM_SHARED((NP, D), jnp.float32),
        pltpu.SemaphoreType.DMA,
    ],
    name="sc_seg_sum",
)


def _layer1_body(acc_ref, x_ref, deg_ref, w_ref, b_ref, o_ref):
  a = acc_ref[0] + acc_ref[1] + x_ref[...]
  deg = jnp.sum(deg_ref[...], axis=1, keepdims=True)
  z = a / (deg + 1.0)
  o_ref[...] = jnp.maximum(
      jnp.dot(z, w_ref[...], preferred_element_type=jnp.float32)
      + b_ref[...], 0.0)


def _layer2_body(acc_ref, h_ref, deg_ref, w2_ref, b2_ref, w3_ref, b3_ref,
                 emb_ref, out_ref):
  a = acc_ref[0] + acc_ref[1] + h_ref[...]
  deg = jnp.sum(deg_ref[...], axis=1, keepdims=True)
  z = a / (deg + 1.0)
  e = jnp.maximum(
      jnp.dot(z, w2_ref[...], preferred_element_type=jnp.float32)
      + b2_ref[...], 0.0)
  emb_ref[...] = e
  out_ref[...] = (
      jnp.dot(e, w3_ref[...], preferred_element_type=jnp.float32)
      + b3_ref[...])


BR = 1000


def _tc_layer1(acc, x, degT, w1, b1):
  grid = (N // BR,)
  return pl.pallas_call(
      _layer1_body,
      grid=grid,
      in_specs=[
          pl.BlockSpec((NC, BR, D), lambda i: (0, i, 0)),
          pl.BlockSpec((BR, D), lambda i: (i, 0)),
          pl.BlockSpec((BR, NW), lambda i: (i, 0)),
          pl.BlockSpec((D, D), lambda i: (0, 0)),
          pl.BlockSpec((1, D), lambda i: (0, 0)),
      ],
      out_specs=pl.BlockSpec((BR, D), lambda i: (i, 0)),
      out_shape=jax.ShapeDtypeStruct((N, D), jnp.float32),
      name="tc_sage_layer1",
  )(acc, x, degT, w1, b1)


def _tc_layer2(acc, h, degT, w2, b2, w3p, b3p):
  grid = (N // BR,)
  return pl.pallas_call(
      _layer2_body,
      grid=grid,
      in_specs=[
          pl.BlockSpec((NC, BR, D), lambda i: (0, i, 0)),
          pl.BlockSpec((BR, D), lambda i: (i, 0)),
          pl.BlockSpec((BR, NW), lambda i: (i, 0)),
          pl.BlockSpec((D, D), lambda i: (0, 0)),
          pl.BlockSpec((1, D), lambda i: (0, 0)),
          pl.BlockSpec((D, D), lambda i: (0, 0)),
          pl.BlockSpec((1, D), lambda i: (0, 0)),
      ],
      out_specs=[
          pl.BlockSpec((BR, D), lambda i: (i, 0)),
          pl.BlockSpec((BR, D), lambda i: (i, 0)),
      ],
      out_shape=[
          jax.ShapeDtypeStruct((N, D), jnp.float32),
          jax.ShapeDtypeStruct((N, D), jnp.float32),
      ],
      name="tc_sage_layer2",
  )(acc, h, degT, w2, b2, w3p, b3p)


def kernel(x, edge_index, W1, b1, W2, b2, W3, b3):
  src = edge_index[0].astype(jnp.int32)
  dst = edge_index[1].astype(jnp.int32)

  zrow = jnp.zeros((CH, D), jnp.float32)
  zdeg = jnp.zeros((NP,), jnp.float32)

  acc1, degp = _seg_sum_deg(src, dst, x, zrow, zdeg)
  degT = degp.reshape(NW, NP).T
  h1 = _tc_layer1(acc1, x, degT, W1, b1.reshape(1, D))

  acc2 = _seg_sum_plain(src, dst, h1, zrow)
  w3p = jnp.zeros((D, D), jnp.float32).at[:, :NCLS].set(W3)
  b3p = jnp.zeros((1, D), jnp.float32).at[0, :NCLS].set(b3)
  emb, outp = _tc_layer2(acc2, h1, degT, W2, b2.reshape(1, D), w3p, b3p)
  return (outp[:, :NCLS], emb)
